# Initial kernel scaffold; baseline (speedup 1.0000x reference)
import jax
import jax.numpy as jnp
from jax.experimental import pallas as pl


def kernel(pred_boxes, gt_boxes, masks):
    raise NotImplementedError("write your pallas kernel here")


# speedup vs baseline: 3.4442x; 3.4442x over previous
import functools

import jax
import jax.numpy as jnp
from jax import lax
from jax.experimental import pallas as pl
from jax.experimental.pallas import tpu as pltpu
from jax.experimental.pallas import tpu_sc as plsc

N, P, M, C = 8, 2048, 512, 4
L = 16
NC, NS = 2, 16
NW = NC * NS
QPW = (N * P) // NW
SUB_PER_IMG = NW // N
GROUP = 8
NVREG = QPW // L

_BIG = 1e30
_SENTINEL = 1e8


def _sc_body(pred_hbm, gt_hbm, out_hbm, pred_v, gt_v, out_v):
    w = lax.axis_index("s") * NC + lax.axis_index("c")
    img = w // SUB_PER_IMG
    pltpu.sync_copy(pred_hbm.at[w], pred_v)
    pltpu.sync_copy(gt_hbm.at[img], gt_v)

    for g in range(NVREG // GROUP):
        px = [[pred_v[k, pl.ds((g * GROUP + j) * L, L)] for k in range(C)]
              for j in range(GROUP)]
        acc0 = tuple(jnp.full((L,), _BIG, jnp.float32) for _ in range(GROUP))

        def body(m, acc, px=px):
            gk = [plsc.load_gather(gt_v, [jnp.full((L,), m + k * M, jnp.int32)])
                  for k in range(C)]
            out = []
            for j in range(GROUP):
                d = jnp.abs(px[j][0] - gk[0])
                for k in range(1, C):
                    d = d + jnp.abs(px[j][k] - gk[k])
                out.append(jnp.minimum(acc[j], d))
            return tuple(out)

        acc = lax.fori_loop(0, M, body, acc0)
        for j in range(GROUP):
            v = acc[j]
            v = jnp.where(v >= _SENTINEL, 0.0, v)
            out_v[pl.ds((g * GROUP + j) * L, L)] = v

    pltpu.sync_copy(out_v, out_hbm.at[w])


@functools.partial(jax.jit, static_argnames=())
def kernel(pred_boxes, gt_boxes, masks):
    gt_adj = jnp.where(masks[:, :, None], gt_boxes,
                       jnp.full_like(gt_boxes, 1e9))
    pred_r = (pred_boxes.reshape(N, SUB_PER_IMG, QPW, C)
              .transpose(0, 1, 3, 2).reshape(NW, C, QPW))
    gt_r = gt_adj.transpose(0, 2, 1).reshape(N, C * M)

    call = pl.kernel(
        _sc_body,
        out_type=jax.ShapeDtypeStruct((NW, QPW), jnp.float32),
        mesh=plsc.VectorSubcoreMesh(core_axis_name="c", subcore_axis_name="s",
                                    num_cores=NC, num_subcores=NS),
        scratch_types=[
            pltpu.VMEM((C, QPW), jnp.float32),
            pltpu.VMEM((C * M,), jnp.float32),
            pltpu.VMEM((QPW,), jnp.float32),
        ],
        compiler_params=pltpu.CompilerParams(needs_layout_passes=False),
    )
    return call(pred_r, gt_r).reshape(N, P)


# speedup vs baseline: 4.5880x; 1.3321x over previous
import functools

import jax
import jax.numpy as jnp
from jax import lax
from jax.experimental import pallas as pl
from jax.experimental.pallas import tpu as pltpu
from jax.experimental.pallas import tpu_sc as plsc

N, P, M, C = 8, 2048, 512, 4
L = 16
NC, NS = 2, 16
NW = NC * NS
SUB_PER_IMG = NW // N

P_SC = 1024
P_TC = P - P_SC
TP = 256

QPW = (N * P_SC) // NW
NVREG = QPW // L
GROUP = 8

_BIG = 1e30
_SENTINEL = 1e8


def _sc_body(pred_hbm, gt_hbm, out_hbm, pred_v, gt_v, out_v):
    w = lax.axis_index("s") * NC + lax.axis_index("c")
    img = w // SUB_PER_IMG
    pltpu.sync_copy(pred_hbm.at[w], pred_v)
    pltpu.sync_copy(gt_hbm.at[img], gt_v)

    for g in range(0, NVREG, GROUP):
        nv = min(GROUP, NVREG - g)
        px = [[pred_v[k, pl.ds((g + j) * L, L)] for k in range(C)]
              for j in range(nv)]
        acc0 = tuple(jnp.full((L,), _BIG, jnp.float32) for _ in range(nv))

        def body(m, acc, px=px, nv=nv):
            gk = [plsc.load_gather(gt_v, [jnp.full((L,), m + k * M, jnp.int32)])
                  for k in range(C)]
            out = []
            for j in range(nv):
                d = jnp.abs(px[j][0] - gk[0])
                for k in range(1, C):
                    d = d + jnp.abs(px[j][k] - gk[k])
                out.append(jnp.minimum(acc[j], d))
            return tuple(out)

        acc = lax.fori_loop(0, M, body, acc0)
        for j in range(nv):
            v = acc[j]
            v = jnp.where(v >= _SENTINEL, 0.0, v)
            out_v[pl.ds((g + j) * L, L)] = v

    pltpu.sync_copy(out_v, out_hbm.at[w])


def _sc_call(pred_r, gt_r):
    return pl.kernel(
        _sc_body,
        out_type=jax.ShapeDtypeStruct((NW, QPW), jnp.float32),
        mesh=plsc.VectorSubcoreMesh(core_axis_name="c", subcore_axis_name="s",
                                    num_cores=NC, num_subcores=NS),
        scratch_types=[
            pltpu.VMEM((C, QPW), jnp.float32),
            pltpu.VMEM((C * M,), jnp.float32),
            pltpu.VMEM((QPW,), jnp.float32),
        ],
        compiler_params=pltpu.CompilerParams(needs_layout_passes=False),
    )(pred_r, gt_r)


def _tc_body(pred_ref, gt_ref, out_ref):
    p = pred_ref[0]
    g = gt_ref[0]
    acc = jnp.abs(p[:, 0:1] - g[0][None, :])
    for k in range(1, C):
        acc = acc + jnp.abs(p[:, k:k + 1] - g[k][None, :])
    m = jnp.min(acc, axis=1)
    out_ref[0, 0] = jnp.where(m >= _SENTINEL, 0.0, m)


def _tc_call(pred_tc, gt_t):
    return pl.pallas_call(
        _tc_body,
        grid=(N, P_TC // TP),
        in_specs=[
            pl.BlockSpec((1, TP, C), lambda n, t: (n, t, 0)),
            pl.BlockSpec((1, C, M), lambda n, t: (n, 0, 0)),
        ],
        out_specs=pl.BlockSpec((1, 1, TP),
                               lambda n, t: (n * (P_TC // TP) + t, 0, 0)),
        out_shape=jax.ShapeDtypeStruct((N * P_TC // TP, 1, TP), jnp.float32),
    )(pred_tc, gt_t).reshape(N, P_TC)


@jax.jit
def kernel(pred_boxes, gt_boxes, masks):
    gt_adj = jnp.where(masks[:, :, None], gt_boxes,
                       jnp.full_like(gt_boxes, 1e9))
    gt_t = gt_adj.transpose(0, 2, 1)
    gt_r = gt_t.reshape(N, C * M)

    pred_sc = (pred_boxes[:, :P_SC, :]
               .reshape(N, SUB_PER_IMG, QPW, C)
               .transpose(0, 1, 3, 2).reshape(NW, C, QPW))
    out_sc = _sc_call(pred_sc, gt_r)

    out_tc = _tc_call(pred_boxes[:, P_SC:, :], gt_t)

    loss_sc = out_sc.reshape(N, P_SC)
    return jnp.concatenate([loss_sc, out_tc], axis=1)
